# baseline (speedup 1.0000x reference)
import jax
import jax.numpy as jnp
from jax import lax
from jax.experimental import pallas as pl
from jax.experimental.pallas import tpu as pltpu
from jax.experimental.pallas import tpu_sc as plsc

N = 10000
E = 320000
D = 128

NC = 2
NS = 16
K = 80
EDGES_PER_TILE = E // (NC * NS)
CHUNKS = EDGES_PER_TILE // K
GROUPS = 5
GCH = CHUNKS // GROUPS
ROWS_PER_TILE = 624
TAIL_ROWS = N - NS * ROWS_PER_TILE


def _make_seg_sum(compute_deg: bool):
    mesh = plsc.VectorSubcoreMesh(
        core_axis_name="c", subcore_axis_name="s", num_cores=NC, num_subcores=NS
    )
    out_type = [
        jax.ShapeDtypeStruct((N, D), jnp.float32),
        jax.ShapeDtypeStruct((N, D), jnp.float32),
    ]
    if compute_deg:
        out_type.append(jax.ShapeDtypeStruct((10240,), jnp.float32))
        out_type.append(jax.ShapeDtypeStruct((10240,), jnp.float32))

    def body(src_hbm, dst_hbm, g_hbm, zrows_hbm, zflat_hbm, ones_hbm, *refs):
        if compute_deg:
            (outa_hbm, outb_hbm, dega_hbm, degb_hbm,
             srcl_v, dstl_v, r0, r1, r2, r3, ones_v,
             acc_sh, deg_sh, g0, g1, g2, g3, s0, s1, s2, s3) = refs
        else:
            (outa_hbm, outb_hbm, srcl_v, dstl_v, r0, r1, r2, r3,
             acc_sh, g0, g1, g2, g3, s0, s1, s2, s3) = refs
        bufs = (r0, r1, r2, r3)
        gsems = (g0, g1, g2, g3)
        ssems = (s0, s1, s2, s3)
        c = lax.axis_index("c")
        s = lax.axis_index("s")
        wid = c * NS + s

        pltpu.sync_copy(zrows_hbm, acc_sh.at[pl.ds(s * ROWS_PER_TILE, ROWS_PER_TILE)])

        @pl.when(s == 0)
        def _():
            pltpu.sync_copy(
                zrows_hbm.at[pl.ds(0, TAIL_ROWS)],
                acc_sh.at[pl.ds(NS * ROWS_PER_TILE, TAIL_ROWS)],
            )

        if compute_deg:
            pltpu.sync_copy(ones_hbm, ones_v)

            @pl.when(s == 0)
            def _():
                pltpu.sync_copy(zflat_hbm, deg_sh)

        plsc.subcore_barrier()

        def gstart(chunk, buf, sem):
            pltpu.async_copy(g_hbm.at[srcl_v.at[pl.ds(chunk * K, K)]], buf, sem)

        def gwait(chunk, buf, sem):
            pltpu.make_async_copy(
                g_hbm.at[srcl_v.at[pl.ds(chunk * K, K)]], buf, sem).wait()

        def sstart(chunk, buf, sem):
            pltpu.async_copy(buf, acc_sh.at[dstl_v.at[chunk]], sem, add=True)
            if compute_deg:
                pltpu.sync_copy(ones_v, deg_sh.at[dstl_v.at[chunk]], add=True)

        def swait(chunk, buf, sem):
            pltpu.make_async_copy(buf, acc_sh.at[dstl_v.at[chunk]], sem).wait()

        def group(g, carry):
            pltpu.sync_copy(
                src_hbm.at[pl.ds(wid * EDGES_PER_TILE + g * (GCH * K), GCH * K)],
                srcl_v)
            pltpu.sync_copy(dst_hbm.at[wid, g], dstl_v)
            for i in range(3):
                gstart(i, bufs[i], gsems[i])
            for i in range(GCH):
                j = i % 4
                gwait(i, bufs[j], gsems[j])
                sstart(i, bufs[j], ssems[j])
                nxt = i + 3
                if nxt < GCH:
                    jn = nxt % 4
                    if nxt >= 4:
                        swait(nxt - 4, bufs[jn], ssems[jn])
                    gstart(nxt, bufs[jn], gsems[jn])
            for i in range(GCH - 4, GCH):
                swait(i, bufs[i % 4], ssems[i % 4])
            return carry

        for g in range(GROUPS):
            group(g, 0)
        plsc.subcore_barrier()

        @pl.when(c == 0)
        def _():
            pltpu.sync_copy(
                acc_sh.at[pl.ds(s * ROWS_PER_TILE, ROWS_PER_TILE)],
                outa_hbm.at[pl.ds(s * ROWS_PER_TILE, ROWS_PER_TILE)],
            )

        @pl.when(c == 1)
        def _():
            pltpu.sync_copy(
                acc_sh.at[pl.ds(s * ROWS_PER_TILE, ROWS_PER_TILE)],
                outb_hbm.at[pl.ds(s * ROWS_PER_TILE, ROWS_PER_TILE)],
            )

        @pl.when(jnp.logical_and(c == 0, s == 0))
        def _():
            pltpu.sync_copy(
                acc_sh.at[pl.ds(NS * ROWS_PER_TILE, TAIL_ROWS)],
                outa_hbm.at[pl.ds(NS * ROWS_PER_TILE, TAIL_ROWS)],
            )
            if compute_deg:
                pltpu.sync_copy(deg_sh, dega_hbm)

        @pl.when(jnp.logical_and(c == 1, s == 0))
        def _():
            pltpu.sync_copy(
                acc_sh.at[pl.ds(NS * ROWS_PER_TILE, TAIL_ROWS)],
                outb_hbm.at[pl.ds(NS * ROWS_PER_TILE, TAIL_ROWS)],
            )
            if compute_deg:
                pltpu.sync_copy(deg_sh, degb_hbm)

    scratch = [
        pltpu.VMEM((GCH * K,), jnp.int32),
        pltpu.VMEM((GCH, K), jnp.int32),
        pltpu.VMEM((K, D), jnp.float32),
        pltpu.VMEM((K, D), jnp.float32),
        pltpu.VMEM((K, D), jnp.float32),
        pltpu.VMEM((K, D), jnp.float32),
    ]
    if compute_deg:
        scratch.append(pltpu.VMEM((K,), jnp.float32))
    scratch.append(pltpu.VMEM_SHARED((N, D), jnp.float32))
    if compute_deg:
        scratch.append(pltpu.VMEM_SHARED((10240,), jnp.float32))
    scratch.extend([pltpu.SemaphoreType.DMA] * 8)

    return pl.kernel(
        body,
        out_type=out_type,
        mesh=mesh,
        scratch_types=scratch,
        name="seg_sum_deg" if compute_deg else "seg_sum",
    )


_seg_sum_deg = _make_seg_sum(True)
_seg_sum = _make_seg_sum(False)

BM = 2048
_GRID = (N + BM - 1) // BM
N_PAD = BM * _GRID


def _norm_mm_body(sa_ref, sb_ref, g_ref, da_ref, db_ref, b_ref, w1_ref,
                  w2_ref, h_ref, g2_ref):
    blk = pl.ds(pl.program_id(0) * BM, BM)
    r = (1.0 / (da_ref[blk] + db_ref[blk] + 1.0)).reshape(BM, 1)
    hn = (sa_ref[...] + sb_ref[...] + g_ref[...]) * r
    t = lax.dot_general(
        hn, w1_ref[...], (((1,), (1,)), ((), ())),
        preferred_element_type=jnp.float32,
    ) + b_ref[...]
    h = jnp.maximum(t, 0.0)
    h_ref[...] = h
    g2_ref[...] = lax.dot_general(
        h, w2_ref[...], (((1,), (1,)), ((), ())),
        preferred_element_type=jnp.float32,
    )


_norm_mm = pl.pallas_call(
    _norm_mm_body,
    grid=(_GRID,),
    in_specs=[
        pl.BlockSpec((BM, D), lambda i: (i, 0)),
        pl.BlockSpec((BM, D), lambda i: (i, 0)),
        pl.BlockSpec((BM, D), lambda i: (i, 0)),
        pl.BlockSpec((N_PAD,), lambda i: (0,)),
        pl.BlockSpec((N_PAD,), lambda i: (0,)),
        pl.BlockSpec((1, D), lambda i: (0, 0)),
        pl.BlockSpec((D, D), lambda i: (0, 0)),
        pl.BlockSpec((D, D), lambda i: (0, 0)),
    ],
    out_specs=[
        pl.BlockSpec((BM, D), lambda i: (i, 0)),
        pl.BlockSpec((BM, D), lambda i: (i, 0)),
    ],
    out_shape=[
        jax.ShapeDtypeStruct((N, D), jnp.float32),
        jax.ShapeDtypeStruct((N, D), jnp.float32),
    ],
)


def _norm_body(sa_ref, sb_ref, g_ref, da_ref, db_ref, b_ref, o_ref):
    blk = pl.ds(pl.program_id(0) * BM, BM)
    r = (1.0 / (da_ref[blk] + db_ref[blk] + 1.0)).reshape(BM, 1)
    o_ref[...] = (sa_ref[...] + sb_ref[...] + g_ref[...]) * r + b_ref[...]


_norm = pl.pallas_call(
    _norm_body,
    grid=(_GRID,),
    in_specs=[
        pl.BlockSpec((BM, D), lambda i: (i, 0)),
        pl.BlockSpec((BM, D), lambda i: (i, 0)),
        pl.BlockSpec((BM, D), lambda i: (i, 0)),
        pl.BlockSpec((N_PAD,), lambda i: (0,)),
        pl.BlockSpec((N_PAD,), lambda i: (0,)),
        pl.BlockSpec((1, D), lambda i: (0, 0)),
    ],
    out_specs=pl.BlockSpec((BM, D), lambda i: (i, 0)),
    out_shape=jax.ShapeDtypeStruct((N, D), jnp.float32),
)


@jax.jit
def kernel(edge_index, feats, W1, b1, W2, b2):
    src = edge_index[0]
    dst = edge_index[1].reshape(NC * NS, GROUPS, GCH, K)
    zrows = jnp.zeros((ROWS_PER_TILE, D), jnp.float32)
    zflat = jnp.zeros((N_PAD,), jnp.float32)
    ones = jnp.ones((K,), jnp.float32)

    s1a, s1b, dega, degb = _seg_sum_deg(src, dst, feats, zrows, zflat, ones)
    h1, g2 = _norm_mm(s1a, s1b, feats, dega, degb, b1.reshape(1, D),
                      W1, W2)
    s2a, s2b = _seg_sum(src, dst, g2, zrows, zflat, ones)
    h2 = _norm(s2a, s2b, g2, dega, degb, b2.reshape(1, D))
    return (h1, h2)


# speedup vs baseline: 1.0024x; 1.0024x over previous
import jax
import jax.numpy as jnp
from jax import lax
from jax.experimental import pallas as pl
from jax.experimental.pallas import tpu as pltpu
from jax.experimental.pallas import tpu_sc as plsc

N = 10000
E = 320000
D = 128

NC = 2
NS = 16
K = 80
EDGES_PER_TILE = E // (NC * NS)
CHUNKS = EDGES_PER_TILE // K
GROUPS = 5
GCH = CHUNKS // GROUPS
ROWS_PER_TILE = 624
TAIL_ROWS = N - NS * ROWS_PER_TILE


def _make_seg_sum(compute_deg: bool):
    mesh = plsc.VectorSubcoreMesh(
        core_axis_name="c", subcore_axis_name="s", num_cores=NC, num_subcores=NS
    )
    out_type = [
        jax.ShapeDtypeStruct((N, D), jnp.float32),
        jax.ShapeDtypeStruct((N, D), jnp.float32),
    ]
    if compute_deg:
        out_type.append(jax.ShapeDtypeStruct((10240,), jnp.float32))
        out_type.append(jax.ShapeDtypeStruct((10240,), jnp.float32))

    def body(src_hbm, dst_hbm, g_hbm, zrows_hbm, zflat_hbm, ones_hbm, *refs):
        if compute_deg:
            (outa_hbm, outb_hbm, dega_hbm, degb_hbm,
             srcl_v, dstl_v, r0, r1, r2, r3, ones_v,
             acc_sh, deg_sh, g0, g1, g2, g3, s0, s1, s2, s3) = refs
        else:
            (outa_hbm, outb_hbm, srcl_v, dstl_v, r0, r1, r2, r3,
             acc_sh, g0, g1, g2, g3, s0, s1, s2, s3) = refs
        bufs = (r0, r1, r2, r3)
        gsems = (g0, g1, g2, g3)
        ssems = (s0, s1, s2, s3)
        c = lax.axis_index("c")
        s = lax.axis_index("s")
        wid = c * NS + s

        pltpu.sync_copy(zrows_hbm, acc_sh.at[pl.ds(s * ROWS_PER_TILE, ROWS_PER_TILE)])

        @pl.when(s == 0)
        def _():
            pltpu.sync_copy(
                zrows_hbm.at[pl.ds(0, TAIL_ROWS)],
                acc_sh.at[pl.ds(NS * ROWS_PER_TILE, TAIL_ROWS)],
            )

        if compute_deg:
            pltpu.sync_copy(ones_hbm, ones_v)

            @pl.when(s == 0)
            def _():
                pltpu.sync_copy(zflat_hbm, deg_sh)

        plsc.subcore_barrier()

        def gstart(chunk, buf, sem):
            pltpu.async_copy(g_hbm.at[srcl_v.at[pl.ds(chunk * K, K)]], buf, sem)

        def gwait(chunk, buf, sem):
            pltpu.make_async_copy(
                g_hbm.at[srcl_v.at[pl.ds(chunk * K, K)]], buf, sem).wait()

        def sstart(chunk, buf, sem):
            pltpu.async_copy(buf, acc_sh.at[dstl_v.at[chunk]], sem, add=True)
            if compute_deg:
                pltpu.sync_copy(ones_v, deg_sh.at[dstl_v.at[chunk]], add=True)

        def swait(chunk, buf, sem):
            pltpu.make_async_copy(buf, acc_sh.at[dstl_v.at[chunk]], sem).wait()

        def group(g, carry):
            pltpu.sync_copy(
                src_hbm.at[pl.ds(wid * EDGES_PER_TILE + g * (GCH * K), GCH * K)],
                srcl_v)
            pltpu.sync_copy(dst_hbm.at[wid, g], dstl_v)
            for i in range(3):
                gstart(i, bufs[i], gsems[i])
            for i in range(GCH):
                j = i % 4
                gwait(i, bufs[j], gsems[j])
                sstart(i, bufs[j], ssems[j])
                nxt = i + 3
                if nxt < GCH:
                    jn = nxt % 4
                    if nxt >= 4:
                        swait(nxt - 4, bufs[jn], ssems[jn])
                    gstart(nxt, bufs[jn], gsems[jn])
            for i in range(GCH - 4, GCH):
                swait(i, bufs[i % 4], ssems[i % 4])
            return carry

        lax.fori_loop(0, GROUPS, group, 0)
        plsc.subcore_barrier()

        @pl.when(c == 0)
        def _():
            pltpu.sync_copy(
                acc_sh.at[pl.ds(s * ROWS_PER_TILE, ROWS_PER_TILE)],
                outa_hbm.at[pl.ds(s * ROWS_PER_TILE, ROWS_PER_TILE)],
            )

        @pl.when(c == 1)
        def _():
            pltpu.sync_copy(
                acc_sh.at[pl.ds(s * ROWS_PER_TILE, ROWS_PER_TILE)],
                outb_hbm.at[pl.ds(s * ROWS_PER_TILE, ROWS_PER_TILE)],
            )

        @pl.when(jnp.logical_and(c == 0, s == 0))
        def _():
            pltpu.sync_copy(
                acc_sh.at[pl.ds(NS * ROWS_PER_TILE, TAIL_ROWS)],
                outa_hbm.at[pl.ds(NS * ROWS_PER_TILE, TAIL_ROWS)],
            )
            if compute_deg:
                pltpu.sync_copy(deg_sh, dega_hbm)

        @pl.when(jnp.logical_and(c == 1, s == 0))
        def _():
            pltpu.sync_copy(
                acc_sh.at[pl.ds(NS * ROWS_PER_TILE, TAIL_ROWS)],
                outb_hbm.at[pl.ds(NS * ROWS_PER_TILE, TAIL_ROWS)],
            )
            if compute_deg:
                pltpu.sync_copy(deg_sh, degb_hbm)

    scratch = [
        pltpu.VMEM((GCH * K,), jnp.int32),
        pltpu.VMEM((GCH, K), jnp.int32),
        pltpu.VMEM((K, D), jnp.float32),
        pltpu.VMEM((K, D), jnp.float32),
        pltpu.VMEM((K, D), jnp.float32),
        pltpu.VMEM((K, D), jnp.float32),
    ]
    if compute_deg:
        scratch.append(pltpu.VMEM((K,), jnp.float32))
    scratch.append(pltpu.VMEM_SHARED((N, D), jnp.float32))
    if compute_deg:
        scratch.append(pltpu.VMEM_SHARED((10240,), jnp.float32))
    scratch.extend([pltpu.SemaphoreType.DMA] * 8)

    return pl.kernel(
        body,
        out_type=out_type,
        mesh=mesh,
        scratch_types=scratch,
        name="seg_sum_deg" if compute_deg else "seg_sum",
    )


_seg_sum_deg = _make_seg_sum(True)
_seg_sum = _make_seg_sum(False)

BM = 2048
_GRID = (N + BM - 1) // BM
N_PAD = BM * _GRID


def _norm_mm_body(sa_ref, sb_ref, g_ref, da_ref, db_ref, b_ref, w1_ref,
                  w2_ref, h_ref, g2_ref):
    blk = pl.ds(pl.program_id(0) * BM, BM)
    r = (1.0 / (da_ref[blk] + db_ref[blk] + 1.0)).reshape(BM, 1)
    hn = (sa_ref[...] + sb_ref[...] + g_ref[...]) * r
    t = lax.dot_general(
        hn, w1_ref[...], (((1,), (1,)), ((), ())),
        preferred_element_type=jnp.float32,
    ) + b_ref[...]
    h = jnp.maximum(t, 0.0)
    h_ref[...] = h
    g2_ref[...] = lax.dot_general(
        h, w2_ref[...], (((1,), (1,)), ((), ())),
        preferred_element_type=jnp.float32,
    )


_norm_mm = pl.pallas_call(
    _norm_mm_body,
    grid=(_GRID,),
    in_specs=[
        pl.BlockSpec((BM, D), lambda i: (i, 0)),
        pl.BlockSpec((BM, D), lambda i: (i, 0)),
        pl.BlockSpec((BM, D), lambda i: (i, 0)),
        pl.BlockSpec((N_PAD,), lambda i: (0,)),
        pl.BlockSpec((N_PAD,), lambda i: (0,)),
        pl.BlockSpec((1, D), lambda i: (0, 0)),
        pl.BlockSpec((D, D), lambda i: (0, 0)),
        pl.BlockSpec((D, D), lambda i: (0, 0)),
    ],
    out_specs=[
        pl.BlockSpec((BM, D), lambda i: (i, 0)),
        pl.BlockSpec((BM, D), lambda i: (i, 0)),
    ],
    out_shape=[
        jax.ShapeDtypeStruct((N, D), jnp.float32),
        jax.ShapeDtypeStruct((N, D), jnp.float32),
    ],
)


def _norm_body(sa_ref, sb_ref, g_ref, da_ref, db_ref, b_ref, o_ref):
    blk = pl.ds(pl.program_id(0) * BM, BM)
    r = (1.0 / (da_ref[blk] + db_ref[blk] + 1.0)).reshape(BM, 1)
    o_ref[...] = (sa_ref[...] + sb_ref[...] + g_ref[...]) * r + b_ref[...]


_norm = pl.pallas_call(
    _norm_body,
    grid=(_GRID,),
    in_specs=[
        pl.BlockSpec((BM, D), lambda i: (i, 0)),
        pl.BlockSpec((BM, D), lambda i: (i, 0)),
        pl.BlockSpec((BM, D), lambda i: (i, 0)),
        pl.BlockSpec((N_PAD,), lambda i: (0,)),
        pl.BlockSpec((N_PAD,), lambda i: (0,)),
        pl.BlockSpec((1, D), lambda i: (0, 0)),
    ],
    out_specs=pl.BlockSpec((BM, D), lambda i: (i, 0)),
    out_shape=jax.ShapeDtypeStruct((N, D), jnp.float32),
)


@jax.jit
def kernel(edge_index, feats, W1, b1, W2, b2):
    src = edge_index[0]
    dst = edge_index[1].reshape(NC * NS, GROUPS, GCH, K)
    zrows = jnp.zeros((ROWS_PER_TILE, D), jnp.float32)
    zflat = jnp.zeros((N_PAD,), jnp.float32)
    ones = jnp.ones((K,), jnp.float32)

    s1a, s1b, dega, degb = _seg_sum_deg(src, dst, feats, zrows, zflat, ones)
    h1, g2 = _norm_mm(s1a, s1b, feats, dega, degb, b1.reshape(1, D),
                      W1, W2)
    s2a, s2b = _seg_sum(src, dst, g2, zrows, zflat, ones)
    h2 = _norm(s2a, s2b, g2, dega, degb, b2.reshape(1, D))
    return (h1, h2)
